# Initial kernel scaffold; baseline (speedup 1.0000x reference)
import functools

import jax
import jax.numpy as jnp
from jax import lax
from jax.experimental import pallas as pl
from jax.experimental.pallas import tpu as pltpu
from jax.experimental.pallas import tpu_sc as plsc

N_NODES = 10000
E_TOT = 320000
H = 128
D_ACC = 144
NC, NS = 2, 16
NW = NC * NS
EW = E_TOT // NW
CH = 80
NCHUNK = EW // CH
STRIPE = N_NODES // NS
ZROWS = 125


def _edge_body(y_hbm, eap_hbm, src_hbm, dst_hbm, out_hbm,
               acc, src_v, dst_v, rows_v, ea_v, msg_v, zb_v, sem):
    cid = lax.axis_index("c")
    sid = lax.axis_index("s")
    wid = sid * NC + cid

    def zb_body(i, _):
        for j in range(D_ACC // 16):
            zb_v[i, pl.ds(j * 16, 16)] = jnp.zeros((16,), jnp.float32)
        return 0
    lax.fori_loop(0, ZROWS, zb_body, 0)
    for r in range(STRIPE // ZROWS):
        pltpu.sync_copy(zb_v, acc.at[pl.ds(sid * STRIPE + r * ZROWS, ZROWS)])

    lane = lax.iota(jnp.int32, 16)
    cnt_one = jnp.where(lane == 0, 1.0, 0.0).astype(jnp.float32)

    def mc_body(i, _):
        msg_v[i, pl.ds(H, 16)] = cnt_one
        return 0
    lax.fori_loop(0, CH, mc_body, 0)

    plsc.subcore_barrier()

    base_w = wid * EW

    def chunk(i, _):
        b = base_w + i * CH
        pltpu.sync_copy(src_hbm.at[pl.ds(b, CH)], src_v)
        pltpu.sync_copy(dst_hbm.at[pl.ds(b, CH)], dst_v)
        pltpu.sync_copy(eap_hbm.at[pl.ds(b, CH)], ea_v)
        pltpu.async_copy(y_hbm.at[src_v], rows_v, sem).wait()

        def ed(c, _):
            for j in range(H // 16):
                r = rows_v[c, pl.ds(j * 16, 16)] + ea_v[c, pl.ds(j * 16, 16)]
                msg_v[c, pl.ds(j * 16, 16)] = jnp.where(r > 0.0, r, 0.2 * r)
            return 0
        lax.fori_loop(0, CH, ed, 0)

        pltpu.sync_copy(msg_v, acc.at[dst_v], add=True)
        return 0
    lax.fori_loop(0, NCHUNK, chunk, 0)

    plsc.subcore_barrier()
    pltpu.sync_copy(acc.at[pl.ds(sid * STRIPE, STRIPE)],
                    out_hbm.at[cid, pl.ds(sid * STRIPE, STRIPE)])


_edge_call = functools.partial(
    pl.kernel,
    out_type=jax.ShapeDtypeStruct((NC, N_NODES, D_ACC), jnp.float32),
    mesh=plsc.VectorSubcoreMesh(core_axis_name="c", subcore_axis_name="s"),
    scratch_types=[
        pltpu.VMEM_SHARED((N_NODES, D_ACC), jnp.float32),
        pltpu.VMEM((CH,), jnp.int32),
        pltpu.VMEM((CH,), jnp.int32),
        pltpu.VMEM((CH, H), jnp.float32),
        pltpu.VMEM((CH, H), jnp.float32),
        pltpu.VMEM((CH, D_ACC), jnp.float32),
        pltpu.VMEM((ZROWS, D_ACC), jnp.float32),
        pltpu.SemaphoreType.DMA,
    ],
)(_edge_body)


def _mm(x, W, b):
    M, K = x.shape
    Ho = W.shape[1]
    BM = 2000

    def body(x_ref, w_ref, b_ref, o_ref):
        o_ref[...] = jnp.dot(x_ref[...], w_ref[...],
                             preferred_element_type=jnp.float32) + b_ref[...]

    return pl.pallas_call(
        body,
        grid=(M // BM,),
        in_specs=[
            pl.BlockSpec((BM, K), lambda i: (i, 0)),
            pl.BlockSpec((K, Ho), lambda i: (0, 0)),
            pl.BlockSpec((1, Ho), lambda i: (0, 0)),
        ],
        out_specs=pl.BlockSpec((BM, Ho), lambda i: (i, 0)),
        out_shape=jax.ShapeDtypeStruct((M, Ho), jnp.float32),
    )(x, W, b)


def _mm_ea(ea, W1e):
    E, K = ea.shape
    BM = 8000

    def body(a_ref, w_ref, o_ref):
        o_ref[...] = jnp.dot(a_ref[...], w_ref[...],
                             preferred_element_type=jnp.float32)

    return pl.pallas_call(
        body,
        grid=(E // BM,),
        in_specs=[
            pl.BlockSpec((BM, K), lambda i: (i, 0)),
            pl.BlockSpec((K, H), lambda i: (0, 0)),
        ],
        out_specs=pl.BlockSpec((BM, H), lambda i: (i, 0)),
        out_shape=jax.ShapeDtypeStruct((E, H), jnp.float32),
    )(ea, W1e)


def _out_transform(agg, W2s, tprime):
    BM = 2000

    def body(a0_ref, a1_ref, w_ref, t_ref, o_ref):
        s = a0_ref[0] + a1_ref[0]
        feats = s[:, :H]
        cnt = s[:, H:H + 1]
        o_ref[...] = (jnp.dot(feats, w_ref[...],
                              preferred_element_type=jnp.float32)
                      + cnt * t_ref[...])

    return pl.pallas_call(
        body,
        grid=(N_NODES // BM,),
        in_specs=[
            pl.BlockSpec((1, BM, D_ACC), lambda i: (0, i, 0)),
            pl.BlockSpec((1, BM, D_ACC), lambda i: (1, i, 0)),
            pl.BlockSpec((H, H), lambda i: (0, 0)),
            pl.BlockSpec((1, H), lambda i: (0, 0)),
        ],
        out_specs=pl.BlockSpec((BM, H), lambda i: (i, 0)),
        out_shape=jax.ShapeDtypeStruct((N_NODES, H), jnp.float32),
    )(agg, agg, W2s, tprime)


def _classifier(x, W1, b1, W2p, b2p):
    BM = 2000

    def body(x_ref, w1_ref, b1_ref, w2_ref, b2_ref, o_ref):
        h = jnp.dot(x_ref[...], w1_ref[...],
                    preferred_element_type=jnp.float32) + b1_ref[...]
        h = jnp.where(h > 0.0, h, 0.2 * h)
        o_ref[...] = jax.nn.sigmoid(
            jnp.dot(h, w2_ref[...], preferred_element_type=jnp.float32)
            + b2_ref[...])

    return pl.pallas_call(
        body,
        grid=(N_NODES // BM,),
        in_specs=[
            pl.BlockSpec((BM, H), lambda i: (i, 0)),
            pl.BlockSpec((H, H), lambda i: (0, 0)),
            pl.BlockSpec((1, H), lambda i: (0, 0)),
            pl.BlockSpec((H, 128), lambda i: (0, 0)),
            pl.BlockSpec((1, 128), lambda i: (0, 0)),
        ],
        out_specs=pl.BlockSpec((BM, 128), lambda i: (i, 0)),
        out_shape=jax.ShapeDtypeStruct((N_NODES, 128), jnp.float32),
    )(x, W1, b1, W2p, b2p)


def _mpl(x_src, src, dst, ea, pp):
    W1x = pp['W1'][:H]
    W1e = pp['W1'][H:]
    y = _mm(x_src, W1x, pp['b1'].reshape(1, H))
    eap = _mm_ea(ea, W1e)
    agg = _edge_call(y, eap, src, dst)
    s = pp['bn_g'] / jnp.sqrt(pp['bn_rv'] + 1e-5)
    W2s = pp['W2'] * s[None, :]
    tprime = (pp['b2'] - pp['bn_rm']) * s + pp['bn_b']
    return _out_transform(agg, W2s, tprime.reshape(1, H))


def kernel(x_var, x_cons, edge_index, edge_attr, rev_edge_index,
           rev_edge_attr, params):
    p = params
    src_vc = edge_index[0].astype(jnp.int32)
    dst_vc = edge_index[1].astype(jnp.int32)
    src_cv = rev_edge_index[0].astype(jnp.int32)
    dst_cv = rev_edge_index[1].astype(jnp.int32)

    hv = _mm(x_var, p['emb_var_W'], p['emb_var_b'].reshape(1, H))
    hc = _mm(x_cons, p['emb_cons_W'], p['emb_cons_b'].reshape(1, H))

    for lp in p['layers']:
        new_hc = _mpl(hv, src_vc, dst_vc, edge_attr, lp['vc'])
        new_hv = _mpl(hc, src_cv, dst_cv, rev_edge_attr, lp['cv'])
        hv, hc = new_hv, new_hc

    W2p = jnp.zeros((H, 128), jnp.float32).at[:, 0].set(p['cls_W2'][:, 0])
    b2p = jnp.zeros((1, 128), jnp.float32).at[0, 0].set(p['cls_b2'][0])
    out = _classifier(hv, p['cls_W1'], p['cls_b1'].reshape(1, H), W2p, b2p)
    return out[:, 0]


# speedup vs baseline: 2.5849x; 2.5849x over previous
import functools

import jax
import jax.numpy as jnp
from jax import lax
from jax.experimental import pallas as pl
from jax.experimental.pallas import tpu as pltpu
from jax.experimental.pallas import tpu_sc as plsc

N_NODES = 10000
N_PAD = 10240
E_TOT = 320000
H = 128
NC, NS = 2, 16
NW = NC * NS
EW = E_TOT // NW
CH = 80
NCHUNK = EW // CH
STRIPE = N_PAD // NS
ZROWS = 128


def _edge_body(y_hbm, eap_hbm, src_hbm, dst_hbm, out_hbm,
               acc, src_v, dst_v, rows_v, ea_v, msg_v, zb_v, sem):
    cid = lax.axis_index("c")
    sid = lax.axis_index("s")
    wid = sid * NC + cid

    def zb_body(i, _):
        for j in range(H // 16):
            zb_v[i, pl.ds(j * 16, 16)] = jnp.zeros((16,), jnp.float32)
        return 0
    lax.fori_loop(0, ZROWS, zb_body, 0)
    for r in range(STRIPE // ZROWS):
        pltpu.sync_copy(zb_v, acc.at[pl.ds(sid * STRIPE + r * ZROWS, ZROWS)])

    plsc.subcore_barrier()

    base_w = wid * EW

    def chunk(i, _):
        b = base_w + i * CH
        pltpu.sync_copy(src_hbm.at[pl.ds(b, CH)], src_v)
        pltpu.sync_copy(dst_hbm.at[pl.ds(b, CH)], dst_v)
        pltpu.sync_copy(eap_hbm.at[pl.ds(b, CH)], ea_v)
        pltpu.async_copy(y_hbm.at[src_v], rows_v, sem).wait()

        def ed(c, _):
            for j in range(H // 16):
                r = rows_v[c, pl.ds(j * 16, 16)] + ea_v[c, pl.ds(j * 16, 16)]
                msg_v[c, pl.ds(j * 16, 16)] = jnp.where(r > 0.0, r, 0.2 * r)
            return 0
        lax.fori_loop(0, CH, ed, 0)

        pltpu.sync_copy(msg_v, acc.at[dst_v], add=True)
        return 0
    lax.fori_loop(0, NCHUNK, chunk, 0)

    plsc.subcore_barrier()
    pltpu.sync_copy(acc.at[pl.ds(sid * STRIPE, STRIPE)],
                    out_hbm.at[cid, pl.ds(sid * STRIPE, STRIPE)])


_edge_call = functools.partial(
    pl.kernel,
    out_type=jax.ShapeDtypeStruct((NC, N_PAD, H), jnp.float32),
    mesh=plsc.VectorSubcoreMesh(core_axis_name="c", subcore_axis_name="s"),
    scratch_types=[
        pltpu.VMEM_SHARED((N_PAD, H), jnp.float32),
        pltpu.VMEM((CH,), jnp.int32),
        pltpu.VMEM((CH,), jnp.int32),
        pltpu.VMEM((CH, H), jnp.float32),
        pltpu.VMEM((CH, H), jnp.float32),
        pltpu.VMEM((CH, H), jnp.float32),
        pltpu.VMEM((ZROWS, H), jnp.float32),
        pltpu.SemaphoreType.DMA,
    ],
)(_edge_body)


def _count_body(dst_hbm, out_hbm, acc, dst_v, ones_v, zb_v, sem):
    cid = lax.axis_index("c")
    sid = lax.axis_index("s")
    wid = sid * NC + cid

    ones16 = jnp.ones((16,), jnp.float32)
    zeros16 = jnp.zeros((16,), jnp.float32)

    def ones_body(i, _):
        ones_v[pl.ds(i * 16, 16)] = ones16
        return 0
    lax.fori_loop(0, CH // 16, ones_body, 0)

    def zb_body(i, _):
        zb_v[pl.ds(i * 16, 16)] = zeros16
        return 0
    lax.fori_loop(0, N_PAD // 16, zb_body, 0)

    @pl.when(sid == 0)
    def _():
        pltpu.sync_copy(zb_v, acc)

    plsc.subcore_barrier()

    base_w = wid * EW

    def chunk(i, _):
        pltpu.sync_copy(dst_hbm.at[pl.ds(base_w + i * CH, CH)], dst_v)
        pltpu.sync_copy(ones_v, acc.at[dst_v], add=True)
        return 0
    lax.fori_loop(0, NCHUNK, chunk, 0)

    plsc.subcore_barrier()

    @pl.when(sid == 0)
    def _():
        pltpu.sync_copy(acc, out_hbm.at[cid])


_count_call = functools.partial(
    pl.kernel,
    out_type=jax.ShapeDtypeStruct((NC, N_PAD), jnp.float32),
    mesh=plsc.VectorSubcoreMesh(core_axis_name="c", subcore_axis_name="s"),
    scratch_types=[
        pltpu.VMEM_SHARED((N_PAD,), jnp.float32),
        pltpu.VMEM((CH,), jnp.int32),
        pltpu.VMEM((CH,), jnp.float32),
        pltpu.VMEM((N_PAD,), jnp.float32),
        pltpu.SemaphoreType.DMA,
    ],
)(_count_body)


def _mm(x, W, b):
    M, K = x.shape
    Ho = W.shape[1]
    BM = 2000

    def body(x_ref, w_ref, b_ref, o_ref):
        o_ref[...] = jnp.dot(x_ref[...], w_ref[...],
                             preferred_element_type=jnp.float32) + b_ref[...]

    return pl.pallas_call(
        body,
        grid=(M // BM,),
        in_specs=[
            pl.BlockSpec((BM, K), lambda i: (i, 0)),
            pl.BlockSpec((K, Ho), lambda i: (0, 0)),
            pl.BlockSpec((1, Ho), lambda i: (0, 0)),
        ],
        out_specs=pl.BlockSpec((BM, Ho), lambda i: (i, 0)),
        out_shape=jax.ShapeDtypeStruct((M, Ho), jnp.float32),
    )(x, W, b)


def _mm_ea(ea, W1e):
    E, K = ea.shape
    BM = 8000

    def body(a_ref, w_ref, o_ref):
        o_ref[...] = jnp.dot(a_ref[...], w_ref[...],
                             preferred_element_type=jnp.float32)

    return pl.pallas_call(
        body,
        grid=(E // BM,),
        in_specs=[
            pl.BlockSpec((BM, K), lambda i: (i, 0)),
            pl.BlockSpec((K, H), lambda i: (0, 0)),
        ],
        out_specs=pl.BlockSpec((BM, H), lambda i: (i, 0)),
        out_shape=jax.ShapeDtypeStruct((E, H), jnp.float32),
    )(ea, W1e)


def _out_transform(agg, cnt2, W2s, tprime):
    BM = 2000

    def body(a0_ref, a1_ref, c_ref, w_ref, t_ref, o_ref):
        s = a0_ref[0] + a1_ref[0]
        o_ref[...] = (jnp.dot(s, w_ref[...],
                              preferred_element_type=jnp.float32)
                      + c_ref[...] * t_ref[...])

    return pl.pallas_call(
        body,
        grid=(N_NODES // BM,),
        in_specs=[
            pl.BlockSpec((1, BM, H), lambda i: (0, i, 0)),
            pl.BlockSpec((1, BM, H), lambda i: (1, i, 0)),
            pl.BlockSpec((BM, 1), lambda i: (i, 0)),
            pl.BlockSpec((H, H), lambda i: (0, 0)),
            pl.BlockSpec((1, H), lambda i: (0, 0)),
        ],
        out_specs=pl.BlockSpec((BM, H), lambda i: (i, 0)),
        out_shape=jax.ShapeDtypeStruct((N_NODES, H), jnp.float32),
    )(agg, agg, cnt2, W2s, tprime)


def _classifier(x, W1, b1, W2p, b2p):
    BM = 2000

    def body(x_ref, w1_ref, b1_ref, w2_ref, b2_ref, o_ref):
        h = jnp.dot(x_ref[...], w1_ref[...],
                    preferred_element_type=jnp.float32) + b1_ref[...]
        h = jnp.where(h > 0.0, h, 0.2 * h)
        o_ref[...] = jax.nn.sigmoid(
            jnp.dot(h, w2_ref[...], preferred_element_type=jnp.float32)
            + b2_ref[...])

    return pl.pallas_call(
        body,
        grid=(N_NODES // BM,),
        in_specs=[
            pl.BlockSpec((BM, H), lambda i: (i, 0)),
            pl.BlockSpec((H, H), lambda i: (0, 0)),
            pl.BlockSpec((1, H), lambda i: (0, 0)),
            pl.BlockSpec((H, 128), lambda i: (0, 0)),
            pl.BlockSpec((1, 128), lambda i: (0, 0)),
        ],
        out_specs=pl.BlockSpec((BM, 128), lambda i: (i, 0)),
        out_shape=jax.ShapeDtypeStruct((N_NODES, 128), jnp.float32),
    )(x, W1, b1, W2p, b2p)


def _mpl(x_src, src, dst, ea, cnt2, pp):
    W1x = pp['W1'][:H]
    W1e = pp['W1'][H:]
    y = _mm(x_src, W1x, pp['b1'].reshape(1, H))
    eap = _mm_ea(ea, W1e)
    agg = _edge_call(y, eap, src, dst)
    s = pp['bn_g'] / jnp.sqrt(pp['bn_rv'] + 1e-5)
    W2s = pp['W2'] * s[None, :]
    tprime = (pp['b2'] - pp['bn_rm']) * s + pp['bn_b']
    return _out_transform(agg, cnt2, W2s, tprime.reshape(1, H))


def kernel(x_var, x_cons, edge_index, edge_attr, rev_edge_index,
           rev_edge_attr, params):
    p = params
    src_vc = edge_index[0].astype(jnp.int32)
    dst_vc = edge_index[1].astype(jnp.int32)
    src_cv = rev_edge_index[0].astype(jnp.int32)
    dst_cv = rev_edge_index[1].astype(jnp.int32)

    hv = _mm(x_var, p['emb_var_W'], p['emb_var_b'].reshape(1, H))
    hc = _mm(x_cons, p['emb_cons_W'], p['emb_cons_b'].reshape(1, H))

    cvc = _count_call(dst_vc)
    ccv = _count_call(dst_cv)
    cnt_vc = (cvc[0] + cvc[1])[:N_NODES].reshape(N_NODES, 1)
    cnt_cv = (ccv[0] + ccv[1])[:N_NODES].reshape(N_NODES, 1)

    for lp in p['layers']:
        new_hc = _mpl(hv, src_vc, dst_vc, edge_attr, cnt_vc, lp['vc'])
        new_hv = _mpl(hc, src_cv, dst_cv, rev_edge_attr, cnt_cv, lp['cv'])
        hv, hc = new_hv, new_hc

    W2p = jnp.zeros((H, 128), jnp.float32).at[:, 0].set(p['cls_W2'][:, 0])
    b2p = jnp.zeros((1, 128), jnp.float32).at[0, 0].set(p['cls_b2'][0])
    out = _classifier(hv, p['cls_W1'], p['cls_b1'].reshape(1, H), W2p, b2p)
    return out[:, 0]
